# baseline (speedup 1.0000x reference)
import functools

import jax
import jax.numpy as jnp
from jax import lax
from jax.experimental import pallas as pl
from jax.experimental.pallas import tpu as pltpu
from jax.experimental.pallas import tpu_sc as plsc

B = 16384
D = 32
V = 1000000
L = 16
NC, NS = 2, 16
NW = NC * NS
BPW = B // NW
CHUNK = 128
NCHUNK = BPW // CHUNK

TC = (V + 127) // 128
NROW = 4 * TC * 8
FLAT = NROW * 128
NTILE = 2 * 4 * TC
PER_W = 1956
NGRP = PER_W // 4

_MESH = plsc.VectorSubcoreMesh(
    core_axis_name="c", subcore_axis_name="s", num_cores=NC, num_subcores=NS
)


@functools.partial(
    pl.kernel,
    out_type=(jax.ShapeDtypeStruct((NROW, 128), jnp.float32),
              jax.ShapeDtypeStruct((NROW, 128), jnp.float32)),
    mesh=_MESH,
    scratch_types=[
        pltpu.VMEM((2, 4, 8, 128), jnp.float32),
        pltpu.SemaphoreType.DMA,
        pltpu.SemaphoreType.DMA,
        pltpu.SemaphoreType.DMA,
        pltpu.SemaphoreType.DMA,
    ],
)
def _sc_conv(wt_hbm, ht_hbm, wlin_hbm, hlin_hbm,
             bufs, in0, in1, out0, out1):
    wid = lax.axis_index("s") * NC + lax.axis_index("c")
    insems = (in0, in1)
    outsems = (out0, out1)

    def chunk_refs(s):
        s = jnp.where(s >= NTILE, s - NTILE, s)
        tab = s // (4 * TC)
        rem = s - tab * (4 * TC)
        cg = rem // TC
        w = rem - cg * TC
        row0 = pl.multiple_of(rem * 8, 8)
        return tab, cg, w, row0

    def fire_in(grp, bank):
        for q in range(4):
            s = grp * 4 + q + wid * PER_W
            tab, cg, w, _ = chunk_refs(s)
            src_off = pl.multiple_of(w * 128, 128)
            cg8 = pl.multiple_of(cg * 8, 8)

            @pl.when(tab == 0)
            def _():
                pltpu.async_copy(
                    wt_hbm.at[pl.ds(cg8, 8), pl.ds(src_off, 128)],
                    bufs.at[bank, q], insems[bank])

            @pl.when(tab == 1)
            def _():
                pltpu.async_copy(
                    ht_hbm.at[pl.ds(cg8, 8), pl.ds(src_off, 128)],
                    bufs.at[bank, q], insems[bank])

    def drain_in(bank):
        for q in range(4):
            pltpu.make_async_copy(
                wt_hbm.at[pl.ds(0, 8), pl.ds(0, 128)],
                bufs.at[bank, q], insems[bank]).wait()

    def fire_out(grp, bank):
        for q in range(4):
            s = grp * 4 + q + wid * PER_W
            tab, _, _, row0 = chunk_refs(s)

            @pl.when(tab == 0)
            def _():
                pltpu.async_copy(
                    bufs.at[bank, q],
                    wlin_hbm.at[pl.ds(row0, 8)], outsems[bank])

            @pl.when(tab == 1)
            def _():
                pltpu.async_copy(
                    bufs.at[bank, q],
                    hlin_hbm.at[pl.ds(row0, 8)], outsems[bank])

    def drain_out(bank):
        for q in range(4):
            pltpu.make_async_copy(
                bufs.at[bank, q],
                wlin_hbm.at[pl.ds(0, 8)], outsems[bank]).wait()

    fire_in(0, 0)

    def body(p, carry):
        drain_in(0)
        fire_out(2 * p, 0)

        @pl.when(p > 0)
        def _():
            drain_out(1)

        @pl.when(2 * p + 1 < NGRP)
        def _():
            fire_in(2 * p + 1, 1)

        drain_out(0)

        @pl.when(2 * p + 2 < NGRP)
        def _():
            fire_in(2 * p + 2, 0)

        @pl.when(2 * p + 1 < NGRP)
        def _():
            drain_in(1)
            fire_out(2 * p + 1, 1)

        return carry

    lax.fori_loop(0, (NGRP + 1) // 2, body, 0)


@functools.partial(
    pl.kernel,
    out_type=jax.ShapeDtypeStruct((B,), jnp.float32),
    mesh=_MESH,
    scratch_types=[
        pltpu.VMEM((BPW,), jnp.int32),
        pltpu.VMEM((BPW,), jnp.int32),
        pltpu.VMEM((BPW,), jnp.int32),
        pltpu.VMEM((BPW,), jnp.int32),
        pltpu.VMEM((BPW,), jnp.int32),
        pltpu.VMEM((BPW,), jnp.int32),
        pltpu.VMEM((2, 3, BPW), jnp.int32),
        pltpu.VMEM((D, BPW), jnp.float32),
        pltpu.VMEM((D, BPW), jnp.float32),
        pltpu.VMEM((D, BPW), jnp.float32),
        pltpu.VMEM((BPW,), jnp.float32),
        pltpu.SemaphoreType.DMA,
        pltpu.SemaphoreType.DMA,
    ],
    compiler_params=pltpu.CompilerParams(use_tc_tiling_on_sc=False),
)
def _sc_gather(u_hbm, i_hbm, j_hbm, wlin_hbm, hlin_hbm, x_hbm,
               idx_u, idx_i, idx_j, ba_u, ba_i, ba_j, abuf,
               vu, vi, vj, xbuf, sem0, sem1):
    wid = lax.axis_index("s") * NC + lax.axis_index("c")
    base = wid * BPW
    pltpu.sync_copy(u_hbm.at[pl.ds(base, BPW)], idx_u)
    pltpu.sync_copy(i_hbm.at[pl.ds(base, BPW)], idx_i)
    pltpu.sync_copy(j_hbm.at[pl.ds(base, BPW)], idx_j)

    for src, dst in ((idx_u, ba_u), (idx_i, ba_i), (idx_j, ba_j)):
        for k in range(BPW // L):
            v = jnp.clip(src[pl.ds(k * L, L)], 0, V - 1)
            dst[pl.ds(k * L, L)] = ((v >> 7) << 10) | (v & 127)

    sems = (sem0, sem1)
    tabs = ((ba_u, wlin_hbm, vu), (ba_i, hlin_hbm, vi), (ba_j, hlin_hbm, vj))

    def build_fire(d, bank):
        off = (d >> 3) * (TC * 1024) + (d & 7) * 128
        for t, (bar, tab, vals) in enumerate(tabs):
            for k in range(BPW // L):
                abuf[bank, t, pl.ds(k * L, L)] = bar[pl.ds(k * L, L)] + off
        for t, (bar, tab, vals) in enumerate(tabs):
            for c in range(NCHUNK):
                s = pl.ds(c * CHUNK, CHUNK)
                pltpu.async_copy(
                    tab.at[abuf.at[bank, t, s]],
                    vals.at[d, s], sems[bank])

    def drain(d, bank):
        for t, (bar, tab, vals) in enumerate(tabs):
            for c in range(NCHUNK):
                s = pl.ds(c * CHUNK, CHUNK)
                pltpu.make_async_copy(
                    tab.at[abuf.at[bank, t, s]],
                    vals.at[d, s], sems[bank]).wait()

    build_fire(jnp.int32(0), 0)

    def dbody(p, carry):
        d0 = 2 * p
        build_fire(d0 + 1, 1)
        drain(d0, 0)

        @pl.when(d0 + 2 < D)
        def _():
            build_fire(d0 + 2, 0)

        drain(d0 + 1, 1)
        return carry

    lax.fori_loop(0, D // 2, dbody, 0)

    def gbody(g, carry):
        sl = pl.ds(g * L, L)
        acc = vu[0, sl] * (vi[0, sl] - vj[0, sl])
        for d in range(1, D):
            acc = acc + vu[d, sl] * (vi[d, sl] - vj[d, sl])
        xbuf[sl] = acc
        return carry

    lax.fori_loop(0, BPW // L, gbody, 0)
    pltpu.sync_copy(xbuf, x_hbm.at[pl.ds(base, BPW)])


def _loss_body(x_ref, o_ref):
    o_ref[0, 0] = -jnp.sum(jax.nn.log_sigmoid(x_ref[...]))


_loss_call = pl.pallas_call(
    _loss_body,
    out_shape=jax.ShapeDtypeStruct((1, 1), jnp.float32),
    out_specs=pl.BlockSpec(memory_space=pltpu.SMEM),
)


def kernel(u, i, j, W, H):
    u = u.astype(jnp.int32)
    i = i.astype(jnp.int32)
    j = j.astype(jnp.int32)
    wlin, hlin = _sc_conv(W.T, H.T)
    x = _sc_gather(u, i, j,
                   wlin.reshape(FLAT), hlin.reshape(FLAT))
    return _loss_call(x.reshape(B // 128, 128))[0, 0]


# speedup vs baseline: 1.5186x; 1.5186x over previous
import functools

import jax
import jax.numpy as jnp
from jax import lax
from jax.experimental import pallas as pl
from jax.experimental.pallas import tpu as pltpu
from jax.experimental.pallas import tpu_sc as plsc

B = 16384
D = 32
V = 1000000
L = 16
NC, NS = 2, 16
NW = NC * NS
BPW = B // NW
CHUNK = 128
NCHUNK = BPW // CHUNK

TC = (V + 127) // 128
NROW = 4 * TC * 8
FLAT = NROW * 128
KW = 13
SRC_W = KW * 128
NB = TC // KW
NTILE = 2 * 4 * NB
PER_W = 152
NGRP = PER_W // 4

_MESH = plsc.VectorSubcoreMesh(
    core_axis_name="c", subcore_axis_name="s", num_cores=NC, num_subcores=NS
)


@functools.partial(
    pl.kernel,
    out_type=(jax.ShapeDtypeStruct((NROW, 128), jnp.float32),
              jax.ShapeDtypeStruct((NROW, 128), jnp.float32)),
    mesh=_MESH,
    scratch_types=[
        pltpu.VMEM((2, 4, 8, SRC_W), jnp.float32),
        pltpu.SemaphoreType.DMA,
        pltpu.SemaphoreType.DMA,
        pltpu.SemaphoreType.DMA,
        pltpu.SemaphoreType.DMA,
    ],
)
def _sc_conv(wt_hbm, ht_hbm, wlin_hbm, hlin_hbm,
             bufs, in0, in1, out0, out1):
    wid = lax.axis_index("s") * NC + lax.axis_index("c")
    insems = (in0, in1)
    outsems = (out0, out1)

    def chunk_refs(s):
        s = jnp.where(s >= NTILE, s - NTILE, s)
        tab = s // (4 * NB)
        rem = s - tab * (4 * NB)
        cg = rem // NB
        cb = rem - cg * NB
        row0 = pl.multiple_of((cg * TC + cb * KW) * 8, 8)
        return tab, cg, cb, row0

    def fire_in(grp, bank):
        for q in range(4):
            s = grp * 4 + q + wid * PER_W
            tab, cg, cb, _ = chunk_refs(s)
            src_off = pl.multiple_of(cb * SRC_W, 128)
            cg8 = pl.multiple_of(cg * 8, 8)

            @pl.when(tab == 0)
            def _():
                pltpu.async_copy(
                    wt_hbm.at[pl.ds(cg8, 8), pl.ds(src_off, SRC_W)],
                    bufs.at[bank, q], insems[bank])

            @pl.when(tab == 1)
            def _():
                pltpu.async_copy(
                    ht_hbm.at[pl.ds(cg8, 8), pl.ds(src_off, SRC_W)],
                    bufs.at[bank, q], insems[bank])

    def drain_in(bank):
        for q in range(4):
            pltpu.make_async_copy(
                wt_hbm.at[pl.ds(0, 8), pl.ds(0, SRC_W)],
                bufs.at[bank, q], insems[bank]).wait()

    def fire_out(grp, bank):
        for q in range(4):
            s = grp * 4 + q + wid * PER_W
            tab, _, _, row0 = chunk_refs(s)
            for w in range(KW):
                src = bufs.at[bank, q, :, pl.ds(w * 128, 128)]

                @pl.when(tab == 0)
                def _():
                    pltpu.async_copy(
                        src, wlin_hbm.at[pl.ds(pl.multiple_of(row0 + 8 * w, 8), 8)],
                        outsems[bank])

                @pl.when(tab == 1)
                def _():
                    pltpu.async_copy(
                        src, hlin_hbm.at[pl.ds(pl.multiple_of(row0 + 8 * w, 8), 8)],
                        outsems[bank])

    def drain_out(bank):
        for q in range(4):
            for w in range(KW):
                pltpu.make_async_copy(
                    bufs.at[bank, q, :, pl.ds(w * 128, 128)],
                    wlin_hbm.at[pl.ds(0, 8)], outsems[bank]).wait()

    fire_in(0, 0)

    def body(p, carry):
        drain_in(0)
        fire_out(2 * p, 0)

        @pl.when(p > 0)
        def _():
            drain_out(1)

        @pl.when(2 * p + 1 < NGRP)
        def _():
            fire_in(2 * p + 1, 1)

        drain_out(0)

        @pl.when(2 * p + 2 < NGRP)
        def _():
            fire_in(2 * p + 2, 0)

        @pl.when(2 * p + 1 < NGRP)
        def _():
            drain_in(1)
            fire_out(2 * p + 1, 1)

        return carry

    lax.fori_loop(0, (NGRP + 1) // 2, body, 0)
    drain_out(1)


@functools.partial(
    pl.kernel,
    out_type=jax.ShapeDtypeStruct((B,), jnp.float32),
    mesh=_MESH,
    scratch_types=[
        pltpu.VMEM((BPW,), jnp.int32),
        pltpu.VMEM((BPW,), jnp.int32),
        pltpu.VMEM((BPW,), jnp.int32),
        pltpu.VMEM((BPW,), jnp.int32),
        pltpu.VMEM((BPW,), jnp.int32),
        pltpu.VMEM((BPW,), jnp.int32),
        pltpu.VMEM((2, 3, BPW), jnp.int32),
        pltpu.VMEM((D, BPW), jnp.float32),
        pltpu.VMEM((D, BPW), jnp.float32),
        pltpu.VMEM((D, BPW), jnp.float32),
        pltpu.VMEM((BPW,), jnp.float32),
        pltpu.SemaphoreType.DMA,
        pltpu.SemaphoreType.DMA,
    ],
    compiler_params=pltpu.CompilerParams(use_tc_tiling_on_sc=False),
)
def _sc_gather(u_hbm, i_hbm, j_hbm, wlin_hbm, hlin_hbm, x_hbm,
               idx_u, idx_i, idx_j, ba_u, ba_i, ba_j, abuf,
               vu, vi, vj, xbuf, sem0, sem1):
    wid = lax.axis_index("s") * NC + lax.axis_index("c")
    base = wid * BPW
    pltpu.sync_copy(u_hbm.at[pl.ds(base, BPW)], idx_u)
    pltpu.sync_copy(i_hbm.at[pl.ds(base, BPW)], idx_i)
    pltpu.sync_copy(j_hbm.at[pl.ds(base, BPW)], idx_j)

    for src, dst in ((idx_u, ba_u), (idx_i, ba_i), (idx_j, ba_j)):
        for k in range(BPW // L):
            v = jnp.clip(src[pl.ds(k * L, L)], 0, V - 1)
            dst[pl.ds(k * L, L)] = ((v >> 7) << 10) | (v & 127)

    sems = (sem0, sem1)
    tabs = ((ba_u, wlin_hbm, vu), (ba_i, hlin_hbm, vi), (ba_j, hlin_hbm, vj))

    def build_fire(d, bank):
        off = (d >> 3) * (TC * 1024) + (d & 7) * 128
        for t, (bar, tab, vals) in enumerate(tabs):
            for k in range(BPW // L):
                abuf[bank, t, pl.ds(k * L, L)] = bar[pl.ds(k * L, L)] + off
        for t, (bar, tab, vals) in enumerate(tabs):
            for c in range(NCHUNK):
                s = pl.ds(c * CHUNK, CHUNK)
                pltpu.async_copy(
                    tab.at[abuf.at[bank, t, s]],
                    vals.at[d, s], sems[bank])

    def drain(d, bank):
        for t, (bar, tab, vals) in enumerate(tabs):
            for c in range(NCHUNK):
                s = pl.ds(c * CHUNK, CHUNK)
                pltpu.make_async_copy(
                    tab.at[abuf.at[bank, t, s]],
                    vals.at[d, s], sems[bank]).wait()

    build_fire(jnp.int32(0), 0)

    def dbody(p, carry):
        d0 = 2 * p
        build_fire(d0 + 1, 1)
        drain(d0, 0)

        @pl.when(d0 + 2 < D)
        def _():
            build_fire(d0 + 2, 0)

        drain(d0 + 1, 1)
        return carry

    lax.fori_loop(0, D // 2, dbody, 0)

    def gbody(g, carry):
        sl = pl.ds(g * L, L)
        acc = vu[0, sl] * (vi[0, sl] - vj[0, sl])
        for d in range(1, D):
            acc = acc + vu[d, sl] * (vi[d, sl] - vj[d, sl])
        xbuf[sl] = acc
        return carry

    lax.fori_loop(0, BPW // L, gbody, 0)
    pltpu.sync_copy(xbuf, x_hbm.at[pl.ds(base, BPW)])


def _loss_body(x_ref, o_ref):
    o_ref[0, 0] = -jnp.sum(jax.nn.log_sigmoid(x_ref[...]))


_loss_call = pl.pallas_call(
    _loss_body,
    out_shape=jax.ShapeDtypeStruct((1, 1), jnp.float32),
    out_specs=pl.BlockSpec(memory_space=pltpu.SMEM),
)


def kernel(u, i, j, W, H):
    u = u.astype(jnp.int32)
    i = i.astype(jnp.int32)
    j = j.astype(jnp.int32)
    wlin, hlin = _sc_conv(W.T, H.T)
    x = _sc_gather(u, i, j,
                   wlin.reshape(FLAT), hlin.reshape(FLAT))
    return _loss_call(x.reshape(B // 128, 128))[0, 0]


# speedup vs baseline: 1.5216x; 1.0019x over previous
import functools

import jax
import jax.numpy as jnp
from jax import lax
from jax.experimental import pallas as pl
from jax.experimental.pallas import tpu as pltpu
from jax.experimental.pallas import tpu_sc as plsc

B = 16384
D = 32
V = 1000000
L = 16
NC, NS = 2, 16
NW = NC * NS
BPW = B // NW
CHUNK = 128
NCHUNK = BPW // CHUNK

TC = (V + 127) // 128
NROW = 4 * TC * 8
FLAT = NROW * 128
KW = 13
SRC_W = KW * 128
NB = TC // KW
NTILE = 2 * 4 * NB
PER_W = 152
NGRP = PER_W // 4

_MESH = plsc.VectorSubcoreMesh(
    core_axis_name="c", subcore_axis_name="s", num_cores=NC, num_subcores=NS
)


@functools.partial(
    pl.kernel,
    out_type=(jax.ShapeDtypeStruct((NROW, 128), jnp.float32),
              jax.ShapeDtypeStruct((NROW, 128), jnp.float32)),
    mesh=_MESH,
    scratch_types=[
        pltpu.VMEM((2, 4, 8, SRC_W), jnp.float32),
        pltpu.SemaphoreType.DMA,
        pltpu.SemaphoreType.DMA,
        pltpu.SemaphoreType.DMA,
        pltpu.SemaphoreType.DMA,
    ],
)
def _sc_conv(wt_hbm, ht_hbm, wlin_hbm, hlin_hbm,
             bufs, in0, in1, out0, out1):
    wid = lax.axis_index("s") * NC + lax.axis_index("c")
    insems = (in0, in1)
    outsems = (out0, out1)

    def chunk_refs(s):
        s = jnp.where(s >= NTILE, s - NTILE, s)
        tab = s // (4 * NB)
        rem = s - tab * (4 * NB)
        cg = rem // NB
        cb = rem - cg * NB
        row0 = pl.multiple_of((cg * TC + cb * KW) * 8, 8)
        return tab, cg, cb, row0

    def fire_in(grp, bank):
        for q in range(4):
            s = grp * 4 + q + wid * PER_W
            tab, cg, cb, _ = chunk_refs(s)
            src_off = pl.multiple_of(cb * SRC_W, 128)
            cg8 = pl.multiple_of(cg * 8, 8)

            @pl.when(tab == 0)
            def _():
                pltpu.async_copy(
                    wt_hbm.at[pl.ds(cg8, 8), pl.ds(src_off, SRC_W)],
                    bufs.at[bank, q], insems[bank])

            @pl.when(tab == 1)
            def _():
                pltpu.async_copy(
                    ht_hbm.at[pl.ds(cg8, 8), pl.ds(src_off, SRC_W)],
                    bufs.at[bank, q], insems[bank])

    def drain_in(bank):
        for q in range(4):
            pltpu.make_async_copy(
                wt_hbm.at[pl.ds(0, 8), pl.ds(0, SRC_W)],
                bufs.at[bank, q], insems[bank]).wait()

    def fire_out(grp, bank):
        for q in range(4):
            s = grp * 4 + q + wid * PER_W
            tab, _, _, row0 = chunk_refs(s)
            for w in range(KW):
                src = bufs.at[bank, q, :, pl.ds(w * 128, 128)]

                @pl.when(tab == 0)
                def _():
                    pltpu.async_copy(
                        src, wlin_hbm.at[pl.ds(pl.multiple_of(row0 + 8 * w, 8), 8)],
                        outsems[bank])

                @pl.when(tab == 1)
                def _():
                    pltpu.async_copy(
                        src, hlin_hbm.at[pl.ds(pl.multiple_of(row0 + 8 * w, 8), 8)],
                        outsems[bank])

    def drain_out(bank):
        for q in range(4):
            for w in range(KW):
                pltpu.make_async_copy(
                    bufs.at[bank, q, :, pl.ds(w * 128, 128)],
                    wlin_hbm.at[pl.ds(0, 8)], outsems[bank]).wait()

    fire_in(0, 0)

    def body(p, carry):
        drain_in(0)
        fire_out(2 * p, 0)

        @pl.when(p > 0)
        def _():
            drain_out(1)

        @pl.when(2 * p + 1 < NGRP)
        def _():
            fire_in(2 * p + 1, 1)

        drain_out(0)

        @pl.when(2 * p + 2 < NGRP)
        def _():
            fire_in(2 * p + 2, 0)

        @pl.when(2 * p + 1 < NGRP)
        def _():
            drain_in(1)
            fire_out(2 * p + 1, 1)

        return carry

    lax.fori_loop(0, (NGRP + 1) // 2, body, 0)
    drain_out(1)


@functools.partial(
    pl.kernel,
    out_type=jax.ShapeDtypeStruct((B,), jnp.float32),
    mesh=_MESH,
    scratch_types=[
        pltpu.VMEM((BPW,), jnp.int32),
        pltpu.VMEM((BPW,), jnp.int32),
        pltpu.VMEM((BPW,), jnp.int32),
        pltpu.VMEM((BPW,), jnp.int32),
        pltpu.VMEM((BPW,), jnp.int32),
        pltpu.VMEM((BPW,), jnp.int32),
        pltpu.VMEM((2, 3, BPW), jnp.int32),
        pltpu.VMEM((D, BPW), jnp.float32),
        pltpu.VMEM((D, BPW), jnp.float32),
        pltpu.VMEM((D, BPW), jnp.float32),
        pltpu.VMEM((BPW,), jnp.float32),
        pltpu.SemaphoreType.DMA,
        pltpu.SemaphoreType.DMA,
    ],
    compiler_params=pltpu.CompilerParams(use_tc_tiling_on_sc=False),
)
def _sc_gather(u_hbm, i_hbm, j_hbm, wlin_hbm, hlin_hbm, x_hbm,
               idx_u, idx_i, idx_j, ba_u, ba_i, ba_j, abuf,
               vu, vi, vj, xbuf, sem0, sem1):
    wid = lax.axis_index("s") * NC + lax.axis_index("c")
    base = wid * BPW
    pltpu.sync_copy(u_hbm.at[pl.ds(base, BPW)], idx_u)
    pltpu.sync_copy(i_hbm.at[pl.ds(base, BPW)], idx_i)
    pltpu.sync_copy(j_hbm.at[pl.ds(base, BPW)], idx_j)

    for src, dst in ((idx_u, ba_u), (idx_i, ba_i), (idx_j, ba_j)):
        for k in range(BPW // L):
            v = jnp.clip(src[pl.ds(k * L, L)], 0, V - 1)
            dst[pl.ds(k * L, L)] = ((v >> 7) << 10) | (v & 127)

    sems = (sem0, sem1)
    tabs = ((ba_u, wlin_hbm, vu), (ba_i, hlin_hbm, vi), (ba_j, hlin_hbm, vj))

    def build_fire(d, bank):
        off = (d >> 3) * (TC * 1024) + (d & 7) * 128
        for t, (bar, tab, vals) in enumerate(tabs):
            for k in range(BPW // L):
                abuf[bank, t, pl.ds(k * L, L)] = bar[pl.ds(k * L, L)] + off
        for t, (bar, tab, vals) in enumerate(tabs):
            for c in range(NCHUNK):
                s = pl.ds(c * CHUNK, CHUNK)
                pltpu.async_copy(
                    tab.at[abuf.at[bank, t, s]],
                    vals.at[d, s], sems[bank])

    def drain(d, bank):
        for t, (bar, tab, vals) in enumerate(tabs):
            for c in range(NCHUNK):
                s = pl.ds(c * CHUNK, CHUNK)
                pltpu.make_async_copy(
                    tab.at[abuf.at[bank, t, s]],
                    vals.at[d, s], sems[bank]).wait()

    build_fire(jnp.int32(0), 0)
    build_fire(jnp.int32(1), 1)

    def dbody(p, carry):
        d0 = 2 * p
        drain(d0, 0)

        @pl.when(d0 + 2 < D)
        def _():
            build_fire(d0 + 2, 0)

        drain(d0 + 1, 1)

        @pl.when(d0 + 3 < D)
        def _():
            build_fire(d0 + 3, 1)

        return carry

    lax.fori_loop(0, D // 2, dbody, 0)

    def gbody(g, carry):
        sl = pl.ds(g * L, L)
        acc = vu[0, sl] * (vi[0, sl] - vj[0, sl])
        for d in range(1, D):
            acc = acc + vu[d, sl] * (vi[d, sl] - vj[d, sl])
        xbuf[sl] = acc
        return carry

    lax.fori_loop(0, BPW // L, gbody, 0)
    pltpu.sync_copy(xbuf, x_hbm.at[pl.ds(base, BPW)])


def _loss_body(x_ref, o_ref):
    o_ref[0, 0] = -jnp.sum(jax.nn.log_sigmoid(x_ref[...]))


_loss_call = pl.pallas_call(
    _loss_body,
    out_shape=jax.ShapeDtypeStruct((1, 1), jnp.float32),
    out_specs=pl.BlockSpec(memory_space=pltpu.SMEM),
)


def kernel(u, i, j, W, H):
    u = u.astype(jnp.int32)
    i = i.astype(jnp.int32)
    j = j.astype(jnp.int32)
    wlin, hlin = _sc_conv(W.T, H.T)
    x = _sc_gather(u, i, j,
                   wlin.reshape(FLAT), hlin.reshape(FLAT))
    return _loss_call(x.reshape(B // 128, 128))[0, 0]
